# Initial kernel scaffold; baseline (speedup 1.0000x reference)
import jax
import jax.numpy as jnp
from jax.experimental import pallas as pl


def kernel(in_feat, edge_index, W1, b1, W2, b2, Wc1, bc1, Wc2, bc2, W3, b3, W4, b4):
    raise NotImplementedError("write your pallas kernel here")


# speedup vs baseline: 10.0600x; 10.0600x over previous
import functools

import jax
import jax.numpy as jnp
from jax import lax
from jax.experimental import pallas as pl
from jax.experimental.pallas import tpu as pltpu
from jax.experimental.pallas import tpu_sc as plsc

N = 10000
E = 320000
H = 128
B = 50
NCHUNK = E // B
NCORE = 2
NSUB = 16
NW = NCORE * NSUB
CPT = NCHUNK // NW
RPT = 624
TAIL = N - NSUB * RPT
DEGW = 16
HH = H // 2

_mesh = plsc.VectorSubcoreMesh(
    core_axis_name="c", subcore_axis_name="s", num_cores=NCORE, num_subcores=NSUB
)

_SC_PARAMS = pltpu.CompilerParams(use_tc_tiling_on_sc=False)

def _zero_fill(ref, nrows, ncols):
    z16 = jnp.zeros((16,), jnp.float32)
    for r in range(nrows):
        for q in range(ncols // 16):
            ref[r, pl.ds(q * 16, 16)] = z16


def _zero_shared(shared, zb, s):
    def zbody(i, carry):
        pltpu.sync_copy(zb, shared.at[pl.ds(s * RPT + i * 16, 16)])
        return carry

    lax.fori_loop(0, RPT // 16, zbody, 0)

    @pl.when(s == NSUB - 1)
    def _tail():
        pltpu.sync_copy(zb, shared.at[pl.ds(NSUB * RPT, TAIL)])


def _drain_shared(shared, out_ref, s):
    pltpu.sync_copy(shared.at[pl.ds(s * RPT, RPT)], out_ref.at[pl.ds(s * RPT, RPT)])

    @pl.when(s == NSUB - 1)
    def _tail():
        pltpu.sync_copy(
            shared.at[pl.ds(NSUB * RPT, TAIL)],
            out_ref.at[pl.ds(NSUB * RPT, TAIL)],
        )


@functools.partial(
    pl.kernel,
    out_type=jax.ShapeDtypeStruct((NCORE, N, DEGW), jnp.float32),
    mesh=_mesh,
    scratch_types=[
        pltpu.VMEM((CPT, B), jnp.int32),
        pltpu.VMEM((B, DEGW), jnp.float32),
        pltpu.VMEM((16, DEGW), jnp.float32),
        pltpu.VMEM_SHARED((N, DEGW), jnp.float32),
    ],
    compiler_params=_SC_PARAMS,
)
def _sc_deg(dst_hbm, out_hbm, didx, ones_v, zb, shared):
    c = lax.axis_index("c")
    s = lax.axis_index("s")
    w = s * NCORE + c

    _zero_fill(zb, 16, DEGW)
    one = jnp.ones((16,), jnp.float32)
    for r in range(B):
        ones_v[r, :] = one

    _zero_shared(shared, zb, s)
    plsc.subcore_barrier()

    pltpu.sync_copy(dst_hbm.at[pl.ds(w * CPT, CPT)], didx)

    def body(j, carry):
        pltpu.sync_copy(ones_v, shared.at[didx.at[j]], add=True)
        return carry

    lax.fori_loop(0, CPT, body, 0)
    plsc.subcore_barrier()
    _drain_shared(shared, out_hbm.at[c], s)


@functools.partial(
    pl.kernel,
    out_type=jax.ShapeDtypeStruct((NCORE, 2, N, HH), jnp.float32),
    mesh=_mesh,
    scratch_types=[
        pltpu.VMEM((CPT, B), jnp.int32),
        pltpu.VMEM((CPT, B), jnp.int32),
        pltpu.VMEM((2, B, HH), jnp.float32),
        pltpu.VMEM((16, HH), jnp.float32),
        pltpu.VMEM_SHARED((N, HH), jnp.float32),
        pltpu.SemaphoreType.DMA,
    ],
    compiler_params=_SC_PARAMS,
)
def _sc_agg(u_hbm, src_hbm, dst_hbm, out_hbm, sidx, didx, rows, zb, shared, gsem):
    c = lax.axis_index("c")
    s = lax.axis_index("s")
    w = s * NCORE + c

    _zero_fill(zb, 16, HH)
    pltpu.sync_copy(src_hbm.at[pl.ds(w * CPT, CPT)], sidx)
    pltpu.sync_copy(dst_hbm.at[pl.ds(w * CPT, CPT)], didx)

    for half in range(2):
        _zero_shared(shared, zb, s)
        plsc.subcore_barrier()

        uh = u_hbm.at[half]
        pltpu.async_copy(uh.at[sidx.at[0]], rows.at[0], gsem)

        def body(j, carry):
            nxt = j + 1

            @pl.when(nxt < CPT)
            def _issue():
                pltpu.make_async_copy(
                    uh.at[sidx.at[nxt]], rows.at[nxt % 2], gsem
                ).start()

            pltpu.make_async_copy(uh.at[sidx.at[j]], rows.at[j % 2], gsem).wait()
            pltpu.sync_copy(rows.at[j % 2], shared.at[didx.at[j]], add=True)
            return carry

        lax.fori_loop(0, CPT, body, 0)
        plsc.subcore_barrier()
        _drain_shared(shared, out_hbm.at[c, half], s)


BLK = 1000
GRID = N // BLK


def _norm_from_deg(degp_ref):
    deg = degp_ref[0, :, 0:1] + degp_ref[1, :, 0:1]
    return lax.rsqrt(jnp.maximum(deg, 1.0))


def _lrelu(x):
    return jnp.where(x >= 0, x, 0.01 * x)


def _store_halves(u_ref, u):
    u_ref[0] = u[:, :HH]
    u_ref[1] = u[:, HH:]


def _agg_from_parts(ap, norm):
    a = ap[0] + ap[1]
    return jnp.concatenate([a[0], a[1]], axis=1) * norm


def _tc1_body(x_ref, w1, b1, w2, b2, degp, h_ref, u_ref):
    norm = _norm_from_deg(degp)
    h = _lrelu(jnp.dot(x_ref[...], w1[...], preferred_element_type=jnp.float32) + b1[...])
    h = _lrelu(jnp.dot(h, w2[...], preferred_element_type=jnp.float32) + b2[...])
    h_ref[...] = h
    _store_halves(u_ref, h * norm)


def _tc2_body(h_ref, ap, degp, wa, wb, bc, h0_ref, u1_ref):
    norm = _norm_from_deg(degp)
    g0 = _agg_from_parts(ap, norm)
    h0 = (
        jnp.dot(h_ref[...], wa[...], preferred_element_type=jnp.float32)
        - jnp.dot(g0, wb[...], preferred_element_type=jnp.float32)
        + bc[...]
    )
    h0_ref[...] = h0
    _store_halves(u1_ref, h0 * norm)


def _tc3_body(h0_ref, ap, degp, wa, wb, bc, w3a, w3b, b3, w4, b4, out_ref):
    norm = _norm_from_deg(degp)
    g1 = _agg_from_parts(ap, norm)
    h0 = h0_ref[...]
    h1 = (
        jnp.dot(h0, wa[...], preferred_element_type=jnp.float32)
        - jnp.dot(g1, wb[...], preferred_element_type=jnp.float32)
        + bc[...]
    )
    z = _lrelu(
        jnp.dot(h0, w3a[...], preferred_element_type=jnp.float32)
        + jnp.dot(h1, w3b[...], preferred_element_type=jnp.float32)
        + b3[...]
    )
    out_ref[...] = jnp.dot(z, w4[...], preferred_element_type=jnp.float32) + b4[...]


def _row_spec(width):
    return pl.BlockSpec((BLK, width), lambda i: (i, 0))


def _part_spec(width):
    return pl.BlockSpec((NCORE, BLK, width), lambda i: (0, i, 0))


def _half_spec():
    return pl.BlockSpec((2, BLK, HH), lambda i: (0, i, 0))


def _agg_spec():
    return pl.BlockSpec((NCORE, 2, BLK, HH), lambda i: (0, 0, i, 0))


def _full_spec(shape):
    return pl.BlockSpec(shape, lambda i: (0,) * len(shape))


_TC_PARAMS = pltpu.CompilerParams(
    dimension_semantics=("arbitrary",),
)


def kernel(in_feat, edge_index, W1, b1, W2, b2, Wc1, bc1, Wc2, bc2, W3, b3, W4, b4):
    src2d = edge_index[0].reshape(NCHUNK, B)
    dst2d = edge_index[1].reshape(NCHUNK, B)
    b1r = b1.reshape(1, H)
    b2r = b2.reshape(1, H)
    bc1r = bc1.reshape(1, H)
    bc2r = bc2.reshape(1, H)
    b3r = b3.reshape(1, H)
    b4r = b4.reshape(1, 2)

    degp = _sc_deg(dst2d)

    h, u = pl.pallas_call(
        _tc1_body,
        grid=(GRID,),
        in_specs=[
            _row_spec(H),
            _full_spec((H, H)),
            _full_spec((1, H)),
            _full_spec((H, H)),
            _full_spec((1, H)),
            _part_spec(DEGW),
        ],
        out_specs=[_row_spec(H), _half_spec()],
        out_shape=[
            jax.ShapeDtypeStruct((N, H), jnp.float32),
            jax.ShapeDtypeStruct((2, N, HH), jnp.float32),
        ],
        compiler_params=_TC_PARAMS,
    )(in_feat, W1, b1r, W2, b2r, degp)

    ap0 = _sc_agg(u, src2d, dst2d)

    h0, u1 = pl.pallas_call(
        _tc2_body,
        grid=(GRID,),
        in_specs=[
            _row_spec(H),
            _agg_spec(),
            _part_spec(DEGW),
            _full_spec((H, H)),
            _full_spec((H, H)),
            _full_spec((1, H)),
        ],
        out_specs=[_row_spec(H), _half_spec()],
        out_shape=[
            jax.ShapeDtypeStruct((N, H), jnp.float32),
            jax.ShapeDtypeStruct((2, N, HH), jnp.float32),
        ],
        compiler_params=_TC_PARAMS,
    )(h, ap0, degp, Wc1[:H], Wc1[H:], bc1r)

    ap1 = _sc_agg(u1, src2d, dst2d)

    out = pl.pallas_call(
        _tc3_body,
        grid=(GRID,),
        in_specs=[
            _row_spec(H),
            _agg_spec(),
            _part_spec(DEGW),
            _full_spec((H, H)),
            _full_spec((H, H)),
            _full_spec((1, H)),
            _full_spec((H, H)),
            _full_spec((H, H)),
            _full_spec((1, H)),
            _full_spec((H, 2)),
            _full_spec((1, 2)),
        ],
        out_specs=_row_spec(2),
        out_shape=jax.ShapeDtypeStruct((N, 2), jnp.float32),
        compiler_params=_TC_PARAMS,
    )(h0, ap1, degp, Wc2[:H], Wc2[H:], bc2r, W3[:H], W3[H:], b3r, W4, b4r)

    return out


# speedup vs baseline: 16.2623x; 1.6165x over previous
import functools

import jax
import jax.numpy as jnp
from jax import lax
from jax.experimental import pallas as pl
from jax.experimental.pallas import tpu as pltpu
from jax.experimental.pallas import tpu_sc as plsc

N = 10000
E = 320000
H = 128
B = 125
NCHUNK = E // B
NCORE = 2
NSUB = 16
NW = NCORE * NSUB
CPT = NCHUNK // NW
R = 4
NWAVE = CPT // R
RPT = 624
TAIL = N - NSUB * RPT
DEGW = 16
HH = H // 2

_mesh = plsc.VectorSubcoreMesh(
    core_axis_name="c", subcore_axis_name="s", num_cores=NCORE, num_subcores=NSUB
)

_SC_PARAMS = pltpu.CompilerParams(use_tc_tiling_on_sc=False)

def _zero_fill(ref, nrows, ncols):
    z16 = jnp.zeros((16,), jnp.float32)
    for r in range(nrows):
        for q in range(ncols // 16):
            ref[r, pl.ds(q * 16, 16)] = z16


def _zero_shared(shared, zb, s):
    def zbody(i, carry):
        pltpu.sync_copy(zb, shared.at[pl.ds(s * RPT + i * 16, 16)])
        return carry

    lax.fori_loop(0, RPT // 16, zbody, 0)

    @pl.when(s == NSUB - 1)
    def _tail():
        pltpu.sync_copy(zb, shared.at[pl.ds(NSUB * RPT, TAIL)])


def _drain_shared(shared, out_ref, s):
    pltpu.sync_copy(shared.at[pl.ds(s * RPT, RPT)], out_ref.at[pl.ds(s * RPT, RPT)])

    @pl.when(s == NSUB - 1)
    def _tail():
        pltpu.sync_copy(
            shared.at[pl.ds(NSUB * RPT, TAIL)],
            out_ref.at[pl.ds(NSUB * RPT, TAIL)],
        )


@functools.partial(
    pl.kernel,
    out_type=jax.ShapeDtypeStruct((NCORE, N, DEGW), jnp.float32),
    mesh=_mesh,
    scratch_types=[
        pltpu.VMEM((CPT, B), jnp.int32),
        pltpu.VMEM((B, DEGW), jnp.float32),
        pltpu.VMEM((16, DEGW), jnp.float32),
        pltpu.VMEM_SHARED((N, DEGW), jnp.float32),
        pltpu.SemaphoreType.DMA,
    ],
    compiler_params=_SC_PARAMS,
)
def _sc_deg(dst_hbm, out_hbm, didx, ones_v, zb, shared, ssem):
    c = lax.axis_index("c")
    s = lax.axis_index("s")
    w = s * NCORE + c

    _zero_fill(zb, 16, DEGW)
    one = jnp.ones((16,), jnp.float32)
    for r in range(B):
        ones_v[r, :] = one

    _zero_shared(shared, zb, s)
    plsc.subcore_barrier()

    pltpu.sync_copy(dst_hbm.at[pl.ds(w * CPT, CPT)], didx)

    def body(j, carry):
        pltpu.async_copy(ones_v, shared.at[didx.at[j]], ssem, add=True)

        @pl.when(j >= 16)
        def _():
            pltpu.make_async_copy(ones_v, shared.at[didx.at[j]], ssem).wait()

        return carry

    lax.fori_loop(0, CPT, body, 0)
    for _ in range(16):
        pltpu.make_async_copy(ones_v, shared.at[didx.at[0]], ssem).wait()
    plsc.subcore_barrier()
    _drain_shared(shared, out_hbm.at[c], s)


@functools.partial(
    pl.kernel,
    out_type=jax.ShapeDtypeStruct((NCORE, 2, N, HH), jnp.float32),
    mesh=_mesh,
    scratch_types=[
        pltpu.VMEM((CPT, B), jnp.int32),
        pltpu.VMEM((CPT, B), jnp.int32),
        pltpu.VMEM((2, R, B, HH), jnp.float32),
        pltpu.VMEM((16, HH), jnp.float32),
        pltpu.VMEM_SHARED((N, HH), jnp.float32),
        pltpu.SemaphoreType.DMA,
        pltpu.SemaphoreType.DMA,
    ],
    compiler_params=_SC_PARAMS,
)
def _sc_agg(u_hbm, src_hbm, dst_hbm, out_hbm, sidx, didx, rows, zb, shared, gsem, ssem):
    c = lax.axis_index("c")
    s = lax.axis_index("s")
    w = s * NCORE + c

    _zero_fill(zb, 16, HH)
    pltpu.sync_copy(src_hbm.at[pl.ds(w * CPT, CPT)], sidx)
    pltpu.sync_copy(dst_hbm.at[pl.ds(w * CPT, CPT)], didx)

    for half in range(2):
        _zero_shared(shared, zb, s)
        plsc.subcore_barrier()

        uh = u_hbm.at[half]
        for b in range(R):
            pltpu.async_copy(uh.at[sidx.at[b]], rows.at[0, b], gsem)
        for b in range(R):
            pltpu.async_copy(uh.at[sidx.at[R + b]], rows.at[1, b], gsem)

        def wave(wv, carry):
            bank = lax.rem(wv, 2)
            base = wv * R
            for b in range(R):
                pltpu.make_async_copy(
                    uh.at[sidx.at[base + b]], rows.at[bank, b], gsem
                ).wait()
                pltpu.async_copy(
                    rows.at[bank, b], shared.at[didx.at[base + b]], ssem, add=True
                )
            for b in range(R):
                pltpu.make_async_copy(
                    rows.at[bank, b], shared.at[didx.at[base + b]], ssem
                ).wait()

            nxt = wv + 2

            @pl.when(nxt < NWAVE)
            def _refill():
                for b in range(R):
                    pltpu.async_copy(
                        uh.at[sidx.at[nxt * R + b]], rows.at[bank, b], gsem
                    )

            return carry

        lax.fori_loop(0, NWAVE, wave, 0)
        plsc.subcore_barrier()
        _drain_shared(shared, out_hbm.at[c, half], s)


BLK = 1000
GRID = N // BLK


def _norm_from_deg(degp_ref):
    deg = degp_ref[0, :, 0:1] + degp_ref[1, :, 0:1]
    return lax.rsqrt(jnp.maximum(deg, 1.0))


def _lrelu(x):
    return jnp.where(x >= 0, x, 0.01 * x)


def _store_halves(u_ref, u):
    u_ref[0] = u[:, :HH]
    u_ref[1] = u[:, HH:]


def _agg_from_parts(ap, norm):
    a = ap[0] + ap[1]
    return jnp.concatenate([a[0], a[1]], axis=1) * norm


def _tc1_body(x_ref, w1, b1, w2, b2, degp, h_ref, u_ref):
    norm = _norm_from_deg(degp)
    h = _lrelu(jnp.dot(x_ref[...], w1[...], preferred_element_type=jnp.float32) + b1[...])
    h = _lrelu(jnp.dot(h, w2[...], preferred_element_type=jnp.float32) + b2[...])
    h_ref[...] = h
    _store_halves(u_ref, h * norm)


def _tc2_body(h_ref, ap, degp, wa, wb, bc, h0_ref, u1_ref):
    norm = _norm_from_deg(degp)
    g0 = _agg_from_parts(ap, norm)
    h0 = (
        jnp.dot(h_ref[...], wa[...], preferred_element_type=jnp.float32)
        - jnp.dot(g0, wb[...], preferred_element_type=jnp.float32)
        + bc[...]
    )
    h0_ref[...] = h0
    _store_halves(u1_ref, h0 * norm)


def _tc3_body(h0_ref, ap, degp, wa, wb, bc, w3a, w3b, b3, w4, b4, out_ref):
    norm = _norm_from_deg(degp)
    g1 = _agg_from_parts(ap, norm)
    h0 = h0_ref[...]
    h1 = (
        jnp.dot(h0, wa[...], preferred_element_type=jnp.float32)
        - jnp.dot(g1, wb[...], preferred_element_type=jnp.float32)
        + bc[...]
    )
    z = _lrelu(
        jnp.dot(h0, w3a[...], preferred_element_type=jnp.float32)
        + jnp.dot(h1, w3b[...], preferred_element_type=jnp.float32)
        + b3[...]
    )
    out_ref[...] = jnp.dot(z, w4[...], preferred_element_type=jnp.float32) + b4[...]


def _row_spec(width):
    return pl.BlockSpec((BLK, width), lambda i: (i, 0))


def _part_spec(width):
    return pl.BlockSpec((NCORE, BLK, width), lambda i: (0, i, 0))


def _half_spec():
    return pl.BlockSpec((2, BLK, HH), lambda i: (0, i, 0))


def _agg_spec():
    return pl.BlockSpec((NCORE, 2, BLK, HH), lambda i: (0, 0, i, 0))


def _full_spec(shape):
    return pl.BlockSpec(shape, lambda i: (0,) * len(shape))


_TC_PARAMS = pltpu.CompilerParams(
    dimension_semantics=("arbitrary",),
)


def kernel(in_feat, edge_index, W1, b1, W2, b2, Wc1, bc1, Wc2, bc2, W3, b3, W4, b4):
    src2d = edge_index[0].reshape(NCHUNK, B)
    dst2d = edge_index[1].reshape(NCHUNK, B)
    b1r = b1.reshape(1, H)
    b2r = b2.reshape(1, H)
    bc1r = bc1.reshape(1, H)
    bc2r = bc2.reshape(1, H)
    b3r = b3.reshape(1, H)
    b4r = b4.reshape(1, 2)

    degp = _sc_deg(dst2d)

    h, u = pl.pallas_call(
        _tc1_body,
        grid=(GRID,),
        in_specs=[
            _row_spec(H),
            _full_spec((H, H)),
            _full_spec((1, H)),
            _full_spec((H, H)),
            _full_spec((1, H)),
            _part_spec(DEGW),
        ],
        out_specs=[_row_spec(H), _half_spec()],
        out_shape=[
            jax.ShapeDtypeStruct((N, H), jnp.float32),
            jax.ShapeDtypeStruct((2, N, HH), jnp.float32),
        ],
        compiler_params=_TC_PARAMS,
    )(in_feat, W1, b1r, W2, b2r, degp)

    ap0 = _sc_agg(u, src2d, dst2d)

    h0, u1 = pl.pallas_call(
        _tc2_body,
        grid=(GRID,),
        in_specs=[
            _row_spec(H),
            _agg_spec(),
            _part_spec(DEGW),
            _full_spec((H, H)),
            _full_spec((H, H)),
            _full_spec((1, H)),
        ],
        out_specs=[_row_spec(H), _half_spec()],
        out_shape=[
            jax.ShapeDtypeStruct((N, H), jnp.float32),
            jax.ShapeDtypeStruct((2, N, HH), jnp.float32),
        ],
        compiler_params=_TC_PARAMS,
    )(h, ap0, degp, Wc1[:H], Wc1[H:], bc1r)

    ap1 = _sc_agg(u1, src2d, dst2d)

    out = pl.pallas_call(
        _tc3_body,
        grid=(GRID,),
        in_specs=[
            _row_spec(H),
            _agg_spec(),
            _part_spec(DEGW),
            _full_spec((H, H)),
            _full_spec((H, H)),
            _full_spec((1, H)),
            _full_spec((H, H)),
            _full_spec((H, H)),
            _full_spec((1, H)),
            _full_spec((H, 2)),
            _full_spec((1, 2)),
        ],
        out_specs=_row_spec(2),
        out_shape=jax.ShapeDtypeStruct((N, 2), jnp.float32),
        compiler_params=_TC_PARAMS,
    )(h0, ap1, degp, Wc2[:H], Wc2[H:], bc2r, W3[:H], W3[H:], b3r, W4, b4r)

    return out
